# Initial kernel scaffold; baseline (speedup 1.0000x reference)
import jax
import jax.numpy as jnp
from jax.experimental import pallas as pl


def kernel(z_e, W):
    raise NotImplementedError("write your pallas kernel here")


# speedup vs baseline: 1.5458x; 1.5458x over previous
import functools

import jax
import jax.numpy as jnp
from jax import lax
from jax.experimental import pallas as pl
from jax.experimental.pallas import tpu as pltpu
from jax.experimental.pallas import tpu_sc as plsc

K_CODES = 8192
D = 256
BETA = 0.25
ORTH_REG_WEIGHT = 0.01

BT = 512
N_TOKENS = 16384
KC = 1024

SC_CORES = 2
SC_SUBCORES = 16
SC_WORKERS = SC_CORES * SC_SUBCORES
ROWS_PER_WORKER = N_TOKENS // SC_WORKERS
GATHER_CHUNK = 128


def _dist_argmin_body(x_ref, w_ref, ids_ref, num_ref, acc_ref):
    i = pl.program_id(0)
    x = x_ref[...]
    w = w_ref[...]
    ones = jnp.ones((1, D), jnp.float32)
    w_sq = lax.dot_general(ones, w * w, (((1,), (1,)), ((), ())),
                           preferred_element_type=jnp.float32)
    dots = lax.dot_general(x, w, (((1,), (1,)), ((), ())),
                           preferred_element_type=jnp.float32)
    x_sq = jnp.sum(x * x, axis=1, keepdims=True)
    dist = (x_sq + w_sq) - 2.0 * dots
    ids = jnp.argmin(dist, axis=1)
    min_s = jnp.min(dist, axis=1)
    ids_ref[...] = ids[:, None]
    part = jnp.sum(min_s)

    @pl.when(i == 0)
    def _():
        acc_ref[0, 0] = part

    @pl.when(i > 0)
    def _():
        acc_ref[0, 0] = acc_ref[0, 0] + part

    @pl.when(i == pl.num_programs(0) - 1)
    def _():
        num_ref[0, 0] = acc_ref[0, 0]


def _gram_losses_body(w_ref, num_ref, out_ref, m_ref):
    i = pl.program_id(0)
    w = w_ref[...]
    row_sq = jnp.sum(w * w, axis=1, keepdims=True)
    norm = jnp.maximum(jnp.sqrt(row_sq), 1e-12)
    wn = w / norm
    contrib = lax.dot_general(wn, wn, (((0,), (0,)), ((), ())),
                              preferred_element_type=jnp.float32)

    @pl.when(i == 0)
    def _():
        m_ref[...] = contrib

    @pl.when(i > 0)
    def _():
        m_ref[...] = m_ref[...] + contrib

    @pl.when(i == pl.num_programs(0) - 1)
    def _():
        m = m_ref[...]
        fro = jnp.sum(m * m)
        r = lax.broadcasted_iota(jnp.int32, (D, D), 0)
        c = lax.broadcasted_iota(jnp.int32, (D, D), 1)
        tr = jnp.sum(jnp.where(r == c, m, 0.0))
        kf = jnp.float32(K_CODES)
        loss_orth = (fro - 2.0 * tr + kf) / (kf * kf)
        mean_sq = num_ref[0, 0] / jnp.float32(N_TOKENS * D)
        loss_codebook = mean_sq
        loss_commit = mean_sq
        loss_vq = loss_codebook + BETA * loss_commit
        loss_total = loss_vq + ORTH_REG_WEIGHT * loss_orth
        out_ref[0, 0] = loss_vq
        out_ref[0, 1] = loss_codebook
        out_ref[0, 2] = loss_commit
        out_ref[0, 3] = loss_orth
        out_ref[0, 4] = loss_total


def _dist_argmin(flat, w):
    grid = N_TOKENS // BT
    return pl.pallas_call(
        _dist_argmin_body,
        grid=(grid,),
        in_specs=[
            pl.BlockSpec((BT, D), lambda i: (i, 0)),
            pl.BlockSpec((K_CODES, D), lambda i: (0, 0)),
        ],
        out_specs=[
            pl.BlockSpec((BT, 1), lambda i: (i, 0)),
            pl.BlockSpec(memory_space=pltpu.SMEM),
        ],
        out_shape=[
            jax.ShapeDtypeStruct((N_TOKENS, 1), jnp.int32),
            jax.ShapeDtypeStruct((1, 1), jnp.float32),
        ],
        scratch_shapes=[pltpu.SMEM((1, 1), jnp.float32)],
    )(flat, w)


def _gram_losses(w, num):
    grid = K_CODES // KC
    return pl.pallas_call(
        _gram_losses_body,
        grid=(grid,),
        in_specs=[
            pl.BlockSpec((KC, D), lambda i: (i, 0)),
            pl.BlockSpec(memory_space=pltpu.SMEM),
        ],
        out_specs=pl.BlockSpec(memory_space=pltpu.SMEM),
        out_shape=jax.ShapeDtypeStruct((1, 8), jnp.float32),
        scratch_shapes=[pltpu.VMEM((D, D), jnp.float32)],
    )(w, num)


def _sc_gather_body(ids_hbm, w_hbm, out_hbm, idx_v, rows_v, sem):
    wid = lax.axis_index("s") * SC_CORES + lax.axis_index("c")
    base = wid * ROWS_PER_WORKER
    for chunk in range(ROWS_PER_WORKER // GATHER_CHUNK):
        off = base + chunk * GATHER_CHUNK
        pltpu.sync_copy(ids_hbm.at[pl.ds(off, GATHER_CHUNK)], idx_v)
        pltpu.async_copy(w_hbm.at[idx_v], rows_v, sem).wait()
        pltpu.sync_copy(rows_v, out_hbm.at[pl.ds(off, GATHER_CHUNK)])


@functools.cache
def _sc_gather():
    return pl.kernel(
        _sc_gather_body,
        out_type=jax.ShapeDtypeStruct((N_TOKENS, D), jnp.float32),
        mesh=plsc.VectorSubcoreMesh(
            core_axis_name="c", subcore_axis_name="s",
            num_cores=SC_CORES, num_subcores=SC_SUBCORES),
        scratch_types=[
            pltpu.VMEM((GATHER_CHUNK,), jnp.int32),
            pltpu.VMEM((GATHER_CHUNK, D), jnp.float32),
            pltpu.SemaphoreType.DMA,
        ],
    )


def kernel(z_e, W):
    B, N, _ = z_e.shape
    flat = z_e.reshape(B * N, D)
    ids2d, num = _dist_argmin(flat, W)
    ids = ids2d.reshape(B * N)
    losses = _gram_losses(W, num)
    z_q = _sc_gather()(ids, W).reshape(B, N, D)
    z_q_st = z_e + (z_q - z_e)
    loss_vq = losses[0, 0]
    loss_codebook = losses[0, 1]
    loss_commit = losses[0, 2]
    loss_orth = losses[0, 3]
    loss_total = losses[0, 4]
    return (z_q_st, ids.reshape(B, N), loss_vq, loss_codebook,
            loss_commit, loss_orth, loss_total)
